# baseline (speedup 1.0000x reference)
import functools

import jax
import jax.numpy as jnp
from jax import lax
from jax.experimental import pallas as pl
from jax.experimental.pallas import tpu as pltpu
from jax.experimental.pallas import tpu_sc as plsc

NUM_C = 512
NUM_D = 8
DV = 64
B = 16
T = 200
BT = B * T
HALF = NUM_C // 2
HW = HALF * NUM_D
PADW = HW + 16

_HIGH = jax.lax.Precision.HIGHEST


def _dot(a, b):
    return jax.lax.dot_general(a, b, (((1,), (0,)), ((), ())),
                               precision=_HIGH)


def _tc_body(x_idx_smem, r_vmem, X_ref, vr_ref, vbeta_ref,
             WihT_ref, bih_ref, WhhT_ref, bhh_ref,
             W2aLT_ref, W1aT_ref, b1a_ref, W1bT_ref, b1b_ref,
             W2aRT_ref, b2a_ref,
             alpha_out, h_out, base_out, u_out,
             gin_ref, gx_ref, h_ref):
    t = pl.program_id(0)

    @pl.when(t == 0)
    def _setup():
        h_ref[...] = jnp.zeros_like(h_ref)
        u_out[...] = _dot(vbeta_ref[...], W2aLT_ref[...])
        gin_ref[:, DV:] = r_vmem[...] * vr_ref[...]

        def gather_one(i, _):
            idx = x_idx_smem[i // B, i % B]
            gin_ref[pl.ds(i, 1), 0:DV] = X_ref[pl.ds(idx, 1), :]
            return 0

        jax.lax.fori_loop(0, BT, gather_one, 0, unroll=8)
        gx_ref[...] = _dot(gin_ref[...], WihT_ref[...]) + bih_ref[...]
        base_out[...] = _dot(gin_ref[...], W2aRT_ref[...]) + b2a_ref[...]

    @pl.when(t > 0)
    def _step():
        t0 = t - 1
        h = h_ref[...]
        gh = _dot(h, WhhT_ref[...]) + bhh_ref[...]
        gx = gx_ref[pl.ds(t0 * B, B), :]
        r_g = jax.nn.sigmoid(gx[:, 0:DV] + gh[:, 0:DV])
        z_g = jax.nn.sigmoid(gx[:, DV:2 * DV] + gh[:, DV:2 * DV])
        n_g = jnp.tanh(gx[:, 2 * DV:] + r_g * gh[:, 2 * DV:])
        h_new = (1.0 - z_g) * n_g + z_g * h
        h_ref[...] = h_new
        h_out[:, pl.ds(t0, 1), :] = h_new.reshape(B, 1, DV)

        @pl.when(t == T)
        def _alpha():
            h_flat = h_out[...].reshape(BT, DV)
            a1 = jnp.maximum(_dot(h_flat, W1aT_ref[...]) + b1a_ref[...], 0.0)
            alpha_out[...] = _dot(a1, W1bT_ref[...]) + b1b_ref[...]


def _run_tc(x_idx_T, r_T, X, v_r, v_beta, W_ih, b_ih, W_hh, b_hh,
            W1a, b1a, W1b, b1b, W2a, b2a):
    smem = pl.BlockSpec(memory_space=pltpu.MemorySpace.SMEM)
    anyv = pl.BlockSpec(memory_space=pltpu.MemorySpace.VMEM)
    grid_spec = pltpu.PrefetchScalarGridSpec(
        num_scalar_prefetch=0,
        grid=(T + 1,),
        in_specs=[smem] + [anyv] * 15,
        out_specs=[
            pl.BlockSpec((BT, NUM_D), lambda t: (0, 0)),
            pl.BlockSpec((B, T, DV), lambda t: (0, 0, 0)),
            pl.BlockSpec((BT, DV), lambda t: (0, 0)),
            pl.BlockSpec((1, DV), lambda t: (0, 0)),
        ],
        scratch_shapes=[
            pltpu.VMEM((BT, 2 * DV), jnp.float32),
            pltpu.VMEM((BT, 3 * DV), jnp.float32),
            pltpu.VMEM((B, DV), jnp.float32),
        ],
    )
    return pl.pallas_call(
        _tc_body,
        grid_spec=grid_spec,
        out_shape=[
            jax.ShapeDtypeStruct((BT, NUM_D), jnp.float32),
            jax.ShapeDtypeStruct((B, T, DV), jnp.float32),
            jax.ShapeDtypeStruct((BT, DV), jnp.float32),
            jax.ShapeDtypeStruct((1, DV), jnp.float32),
        ],
        compiler_params=pltpu.CompilerParams(
            dimension_semantics=("arbitrary",)),
    )(x_idx_T, r_T, X, v_r.reshape(1, DV), v_beta.reshape(1, DV),
      W_ih.T, b_ih.reshape(1, 3 * DV), W_hh.T, b_hh.reshape(1, 3 * DV),
      W2a[:, :DV].T, W1a.T, b1a.reshape(1, DV), W1b.T, b1b.reshape(1, NUM_D),
      W2a[:, DV:].T, b2a.reshape(1, DV))


_GDN = jax.lax.GatherDimensionNumbers(
    offset_dims=(), collapsed_slice_dims=(0,), start_index_map=(0,))


def _shuf(s, perm):
    return jax.lax.gather(
        s, perm[:, None], _GDN, slice_sizes=(1,),
        mode=jax.lax.GatherScatterMode.PROMISE_IN_BOUNDS)

def _sc_step(t, buf_cur, buf_oth, kc, c_v, d_v, base_v, u_v, w2b_v, b2b_vec,
             iota16, first):
    c_t = c_v[pl.ds(t, 16)][0]
    own = (c_t // HALF) == kc
    cl8 = (c_t - kc * HALF) * NUM_D

    if not first:
        c_p = c_v[pl.ds(t - 1, 16)][0]
        own_p = (c_p // HALF) == kc

        @pl.when(own_p)
        def _catch():
            a0 = ((c_p - kc * HALF) * NUM_D) & ~15
            buf_cur[pl.ds(a0, 16)] = buf_oth[pl.ds(a0, 16)]
            buf_cur[pl.ds(a0 + 16, 16)] = buf_oth[pl.ds(a0 + 16, 16)]

    @pl.when(own)
    def _update():
        d_t = d_v[pl.ds(t, 16)][0]
        beta = buf_cur[pl.ds(cl8 + d_t, 16)][0]
        acts = []
        for k in range(4):
            pre = (base_v[pl.ds(t * DV + k * 16, 16)]
                   + beta * u_v[pl.ds(k * 16, 16)])
            acts.append(jnp.maximum(pre, 0.0))
        nc_vec = jnp.zeros((16,), jnp.float32)
        for j in range(NUM_D):
            s = acts[0] * w2b_v[pl.ds(j * DV, 16)]
            for k in range(1, 4):
                s = s + acts[k] * w2b_v[pl.ds(j * DV + k * 16, 16)]
            for sh in (8, 4, 2, 1):
                s = s + _shuf(s, (iota16 + sh) % 16)
            nc_j = s[0] + b2b_vec[j]
            nc_vec = nc_vec + jnp.where(iota16 == j, nc_j, 0.0)
        w = buf_cur[pl.ds(cl8, 16)]
        buf_cur[pl.ds(cl8, 16)] = jnp.where(iota16 < NUM_D, nc_vec, w)


def _make_sc():
    mesh = plsc.VectorSubcoreMesh(core_axis_name="c", subcore_axis_name="s")

    @functools.partial(
        pl.kernel, mesh=mesh,
        out_type=jax.ShapeDtypeStruct((B * T * NUM_C * NUM_D,), jnp.float32),
        scratch_types=[
            pltpu.VMEM((PADW,), jnp.float32),
            pltpu.VMEM((PADW,), jnp.float32),
            pltpu.VMEM((T * DV,), jnp.float32),
            pltpu.VMEM((DV,), jnp.float32),
            pltpu.VMEM((NUM_D * DV,), jnp.float32),
            pltpu.VMEM((16,), jnp.float32),
            pltpu.VMEM((T + 16,), jnp.int32),
            pltpu.VMEM((T + 16,), jnp.int32),
            pltpu.SemaphoreType.DMA,
            pltpu.SemaphoreType.DMA,
        ],
    )
    def sc_kernel(base_hbm, u_hbm, w2b_hbm, b2b_hbm, c_hbm, d_hbm, out_hbm,
                  ping, pong, base_v, u_v, w2b_v, b2b_v, c_v, d_v,
                  semA, semB):
        kc = lax.axis_index("c")
        kb = lax.axis_index("s")
        pltpu.sync_copy(base_hbm.at[pl.ds(kb * (T * DV), T * DV)], base_v)
        pltpu.sync_copy(u_hbm, u_v)
        pltpu.sync_copy(w2b_hbm, w2b_v)
        pltpu.sync_copy(b2b_hbm, b2b_v)
        pltpu.sync_copy(c_hbm.at[pl.ds(kb * T, T)], c_v.at[pl.ds(0, T)])
        pltpu.sync_copy(d_hbm.at[pl.ds(kb * T, T)], d_v.at[pl.ds(0, T)])

        zeros16 = jnp.zeros((16,), jnp.float32)
        for k in range(PADW // 16):
            ping[pl.ds(k * 16, 16)] = zeros16
            pong[pl.ds(k * 16, 16)] = zeros16

        iota16 = jax.lax.broadcasted_iota(jnp.int32, (16,), 0)
        b2b_vec = b2b_v[pl.ds(0, 16)]
        args = (c_v, d_v, base_v, u_v, w2b_v, b2b_vec, iota16)
        off = kc * HW

        _sc_step(0, ping, pong, kc, *args, first=True)
        pltpu.async_copy(ping.at[pl.ds(0, HW)],
                         out_hbm.at[pl.ds((kb * T + (0)) * (NUM_C * NUM_D) + off, HW)], semA)
        _sc_step(1, pong, ping, kc, *args, first=False)
        pltpu.async_copy(pong.at[pl.ds(0, HW)],
                         out_hbm.at[pl.ds((kb * T + (1)) * (NUM_C * NUM_D) + off, HW)], semB)

        def body(i, _):
            t0 = 2 * i
            pltpu.make_async_copy(ping.at[pl.ds(0, HW)],
                                  out_hbm.at[pl.ds((kb * T + (t0)) * (NUM_C * NUM_D) + off, HW)],
                                  semA).wait()
            _sc_step(t0, ping, pong, kc, *args, first=False)
            pltpu.async_copy(ping.at[pl.ds(0, HW)],
                             out_hbm.at[pl.ds((kb * T + (t0)) * (NUM_C * NUM_D) + off, HW)], semA)
            pltpu.make_async_copy(pong.at[pl.ds(0, HW)],
                                  out_hbm.at[pl.ds((kb * T + (t0 + 1)) * (NUM_C * NUM_D) + off, HW)],
                                  semB).wait()
            _sc_step(t0 + 1, pong, ping, kc, *args, first=False)
            pltpu.async_copy(pong.at[pl.ds(0, HW)],
                             out_hbm.at[pl.ds((kb * T + (t0 + 1)) * (NUM_C * NUM_D) + off, HW)], semB)
            return 0

        lax.fori_loop(1, T // 2, body, 0)
        pltpu.make_async_copy(ping.at[pl.ds(0, HW)],
                              out_hbm.at[pl.ds((kb * T + (T - 2)) * (NUM_C * NUM_D) + off, HW)],
                              semA).wait()
        pltpu.make_async_copy(pong.at[pl.ds(0, HW)],
                              out_hbm.at[pl.ds((kb * T + (T - 1)) * (NUM_C * NUM_D) + off, HW)],
                              semB).wait()

    return sc_kernel


def kernel(c_seq, d_seq, r_seq, X, v_r, v_beta, W_ih, W_hh, b_ih, b_hh,
           W1a, b1a, W1b, b1b, W2a, b2a, W2b, b2b):
    c_seq = c_seq.astype(jnp.int32)
    d_seq = d_seq.astype(jnp.int32)
    x_idx_T = (c_seq + NUM_C * d_seq).T
    r_T = r_seq.T.reshape(BT, 1)

    alpha_flat, h_seq, base_tb, u_row = _run_tc(
        x_idx_T, r_T, X, v_r, v_beta, W_ih, b_ih, W_hh, b_hh,
        W1a, b1a, W1b, b1b, W2a, b2a)

    base_bt = jnp.swapaxes(base_tb.reshape(T, B, DV), 0, 1).reshape(B, T * DV)

    sc = _make_sc()
    b2b_pad = jnp.concatenate([b2b, jnp.zeros((16 - NUM_D,), jnp.float32)])
    c_flat = sc(base_bt.reshape(-1), u_row.reshape(DV),
                W2b.reshape(NUM_D * DV), b2b_pad,
                c_seq.reshape(-1), d_seq.reshape(-1))
    C_seq = c_flat.reshape(B, T, NUM_C, NUM_D)

    alpha_seq = alpha_flat.reshape(B, T, NUM_D)
    return (alpha_seq, h_seq, C_seq)


# speedup vs baseline: 4.0636x; 4.0636x over previous
import functools

import jax
import jax.numpy as jnp
from jax import lax
from jax.experimental import pallas as pl
from jax.experimental.pallas import tpu as pltpu
from jax.experimental.pallas import tpu_sc as plsc

NUM_C = 512
NUM_D = 8
DV = 64
B = 16
T = 200
BT = B * T
HALF = NUM_C // 2
HW = HALF * NUM_D
PADW = HW

_HIGH = jax.lax.Precision.HIGHEST


def _dot(a, b):
    return jax.lax.dot_general(a, b, (((1,), (0,)), ((), ())),
                               precision=_HIGH)


def _tc_body(x_idx_smem, r_vmem, X_ref, vr_ref, vbeta_ref,
             WihT_ref, bih_ref, WhhT_ref, bhh_ref,
             W2aLT_ref, W1aT_ref, b1a_ref, W1bT_ref, b1b_ref,
             W2aRT_ref, b2a_ref,
             alpha_out, h_out, base_out, u_out,
             gin_ref, gx_ref, h_ref):
    t = pl.program_id(0)

    @pl.when(t == 0)
    def _setup():
        h_ref[...] = jnp.zeros_like(h_ref)
        u_out[...] = _dot(vbeta_ref[...], W2aLT_ref[...])
        gin_ref[:, DV:] = r_vmem[...] * vr_ref[...]

        def gather_one(i, _):
            idx = x_idx_smem[i // B, i % B]
            gin_ref[pl.ds(i, 1), 0:DV] = X_ref[pl.ds(idx, 1), :]
            return 0

        jax.lax.fori_loop(0, BT, gather_one, 0, unroll=8)
        gx_ref[...] = _dot(gin_ref[...], WihT_ref[...]) + bih_ref[...]
        base_out[...] = _dot(gin_ref[...], W2aRT_ref[...]) + b2a_ref[...]

    @pl.when(t > 0)
    def _step():
        t0 = t - 1
        h = h_ref[...]
        gh = _dot(h, WhhT_ref[...]) + bhh_ref[...]
        gx = gx_ref[pl.ds(t0 * B, B), :]
        r_g = jax.nn.sigmoid(gx[:, 0:DV] + gh[:, 0:DV])
        z_g = jax.nn.sigmoid(gx[:, DV:2 * DV] + gh[:, DV:2 * DV])
        n_g = jnp.tanh(gx[:, 2 * DV:] + r_g * gh[:, 2 * DV:])
        h_new = (1.0 - z_g) * n_g + z_g * h
        h_ref[...] = h_new
        h_out[:, pl.ds(t0, 1), :] = h_new.reshape(B, 1, DV)

        @pl.when(t == T)
        def _alpha():
            h_flat = h_out[...].reshape(BT, DV)
            a1 = jnp.maximum(_dot(h_flat, W1aT_ref[...]) + b1a_ref[...], 0.0)
            alpha_out[...] = _dot(a1, W1bT_ref[...]) + b1b_ref[...]


def _run_tc(x_idx_T, r_T, X, v_r, v_beta, W_ih, b_ih, W_hh, b_hh,
            W1a, b1a, W1b, b1b, W2a, b2a):
    smem = pl.BlockSpec(memory_space=pltpu.MemorySpace.SMEM)
    anyv = pl.BlockSpec(memory_space=pltpu.MemorySpace.VMEM)
    grid_spec = pltpu.PrefetchScalarGridSpec(
        num_scalar_prefetch=0,
        grid=(T + 1,),
        in_specs=[smem] + [anyv] * 15,
        out_specs=[
            pl.BlockSpec((BT, NUM_D), lambda t: (0, 0)),
            pl.BlockSpec((B, T, DV), lambda t: (0, 0, 0)),
            pl.BlockSpec((BT, DV), lambda t: (0, 0)),
            pl.BlockSpec((1, DV), lambda t: (0, 0)),
        ],
        scratch_shapes=[
            pltpu.VMEM((BT, 2 * DV), jnp.float32),
            pltpu.VMEM((BT, 3 * DV), jnp.float32),
            pltpu.VMEM((B, DV), jnp.float32),
        ],
    )
    return pl.pallas_call(
        _tc_body,
        grid_spec=grid_spec,
        out_shape=[
            jax.ShapeDtypeStruct((BT, NUM_D), jnp.float32),
            jax.ShapeDtypeStruct((B, T, DV), jnp.float32),
            jax.ShapeDtypeStruct((BT, DV), jnp.float32),
            jax.ShapeDtypeStruct((1, DV), jnp.float32),
        ],
        compiler_params=pltpu.CompilerParams(
            dimension_semantics=("arbitrary",)),
    )(x_idx_T, r_T, X, v_r.reshape(1, DV), v_beta.reshape(1, DV),
      W_ih.T, b_ih.reshape(1, 3 * DV), W_hh.T, b_hh.reshape(1, 3 * DV),
      W2a[:, :DV].T, W1a.T, b1a.reshape(1, DV), W1b.T, b1b.reshape(1, NUM_D),
      W2a[:, DV:].T, b2a.reshape(1, DV))


_GDN = jax.lax.GatherDimensionNumbers(
    offset_dims=(), collapsed_slice_dims=(0,), start_index_map=(0,))


def _shuf(s, perm):
    return jax.lax.gather(
        s, perm[:, None], _GDN, slice_sizes=(1,),
        mode=jax.lax.GatherScatterMode.PROMISE_IN_BOUNDS)

def _sc_step(t, buf_cur, buf_oth, kc, c_v, d_v, base_v, u_v, w2b_v, b2b_vec,
             iota16, first):
    c_t = c_v[pl.ds(t, 16)][0]
    own = (c_t // HALF) == kc
    c_loc = c_t - kc * HALF
    p0 = (c_loc // 128) * 1024 + (c_loc % 128)
    a0 = p0 & ~15
    lane = p0 & 15

    if not first:
        c_p = c_v[pl.ds(t - 1, 16)][0]
        own_p = (c_p // HALF) == kc

        @pl.when(own_p)
        def _catch():
            cp_loc = c_p - kc * HALF
            b0 = ((cp_loc // 128) * 1024 + (cp_loc % 128)) & ~15
            for j in range(NUM_D):
                buf_cur[pl.ds(b0 + j * 128, 16)] =                     buf_oth[pl.ds(b0 + j * 128, 16)]

    @pl.when(own)
    def _update():
        d_t = d_v[pl.ds(t, 16)][0]
        bw = buf_cur[pl.ds(a0 + d_t * 128, 16)]
        beta = _shuf(bw, jnp.broadcast_to(lane, (16,)))
        acts = []
        for k in range(4):
            pre = (base_v[pl.ds(t * DV + k * 16, 16)]
                   + beta * u_v[pl.ds(k * 16, 16)])
            acts.append(jnp.maximum(pre, 0.0))
        for j in range(NUM_D):
            s = acts[0] * w2b_v[pl.ds(j * DV, 16)]
            for k in range(1, 4):
                s = s + acts[k] * w2b_v[pl.ds(j * DV + k * 16, 16)]
            for sh in (8, 4, 2, 1):
                s = s + _shuf(s, (iota16 + sh) % 16)
            nc_j = s + b2b_vec[j]
            w = buf_cur[pl.ds(a0 + j * 128, 16)]
            buf_cur[pl.ds(a0 + j * 128, 16)] = jnp.where(
                iota16 == lane, nc_j, w)


def _make_sc():
    mesh = plsc.VectorSubcoreMesh(core_axis_name="c", subcore_axis_name="s")

    @functools.partial(
        pl.kernel, mesh=mesh,
        out_type=jax.ShapeDtypeStruct((B * T * NUM_C * NUM_D,), jnp.float32),
        scratch_types=[
            pltpu.VMEM((PADW,), jnp.float32),
            pltpu.VMEM((PADW,), jnp.float32),
            pltpu.VMEM((T * DV,), jnp.float32),
            pltpu.VMEM((DV,), jnp.float32),
            pltpu.VMEM((NUM_D * DV,), jnp.float32),
            pltpu.VMEM((16,), jnp.float32),
            pltpu.VMEM((T + 16,), jnp.int32),
            pltpu.VMEM((T + 16,), jnp.int32),
            pltpu.SemaphoreType.DMA,
            pltpu.SemaphoreType.DMA,
        ],
    )
    def sc_kernel(base_hbm, u_hbm, w2b_hbm, b2b_hbm, c_hbm, d_hbm, out_hbm,
                  ping, pong, base_v, u_v, w2b_v, b2b_v, c_v, d_v,
                  semA, semB):
        kc = lax.axis_index("c")
        kb = lax.axis_index("s")
        pltpu.sync_copy(base_hbm.at[pl.ds(kb * (T * DV), T * DV)], base_v)
        pltpu.sync_copy(u_hbm, u_v)
        pltpu.sync_copy(w2b_hbm, w2b_v)
        pltpu.sync_copy(b2b_hbm, b2b_v)
        pltpu.sync_copy(c_hbm.at[pl.ds(kb * T, T)], c_v.at[pl.ds(0, T)])
        pltpu.sync_copy(d_hbm.at[pl.ds(kb * T, T)], d_v.at[pl.ds(0, T)])

        zeros16 = jnp.zeros((16,), jnp.float32)
        for k in range(PADW // 16):
            ping[pl.ds(k * 16, 16)] = zeros16
            pong[pl.ds(k * 16, 16)] = zeros16

        iota16 = jax.lax.broadcasted_iota(jnp.int32, (16,), 0)
        b2b_vec = b2b_v[pl.ds(0, 16)]
        args = (c_v, d_v, base_v, u_v, w2b_v, b2b_vec, iota16)
        off = kc * HW

        _sc_step(0, ping, pong, kc, *args, first=True)
        pltpu.async_copy(ping.at[pl.ds(0, HW)],
                         out_hbm.at[pl.ds((kb * T + (0)) * (NUM_C * NUM_D) + off, HW)], semA)
        _sc_step(1, pong, ping, kc, *args, first=False)
        pltpu.async_copy(pong.at[pl.ds(0, HW)],
                         out_hbm.at[pl.ds((kb * T + (1)) * (NUM_C * NUM_D) + off, HW)], semB)

        def body(i, _):
            t0 = 2 * i
            pltpu.make_async_copy(ping.at[pl.ds(0, HW)],
                                  out_hbm.at[pl.ds((kb * T + (t0)) * (NUM_C * NUM_D) + off, HW)],
                                  semA).wait()
            _sc_step(t0, ping, pong, kc, *args, first=False)
            pltpu.async_copy(ping.at[pl.ds(0, HW)],
                             out_hbm.at[pl.ds((kb * T + (t0)) * (NUM_C * NUM_D) + off, HW)], semA)
            pltpu.make_async_copy(pong.at[pl.ds(0, HW)],
                                  out_hbm.at[pl.ds((kb * T + (t0 + 1)) * (NUM_C * NUM_D) + off, HW)],
                                  semB).wait()
            _sc_step(t0 + 1, pong, ping, kc, *args, first=False)
            pltpu.async_copy(pong.at[pl.ds(0, HW)],
                             out_hbm.at[pl.ds((kb * T + (t0 + 1)) * (NUM_C * NUM_D) + off, HW)], semB)
            return 0

        lax.fori_loop(1, T // 2, body, 0)
        pltpu.make_async_copy(ping.at[pl.ds(0, HW)],
                              out_hbm.at[pl.ds((kb * T + (T - 2)) * (NUM_C * NUM_D) + off, HW)],
                              semA).wait()
        pltpu.make_async_copy(pong.at[pl.ds(0, HW)],
                              out_hbm.at[pl.ds((kb * T + (T - 1)) * (NUM_C * NUM_D) + off, HW)],
                              semB).wait()

    return sc_kernel


def kernel(c_seq, d_seq, r_seq, X, v_r, v_beta, W_ih, W_hh, b_ih, b_hh,
           W1a, b1a, W1b, b1b, W2a, b2a, W2b, b2b):
    c_seq = c_seq.astype(jnp.int32)
    d_seq = d_seq.astype(jnp.int32)
    x_idx_T = (c_seq + NUM_C * d_seq).T
    r_T = r_seq.T.reshape(BT, 1)

    alpha_flat, h_seq, base_tb, u_row = _run_tc(
        x_idx_T, r_T, X, v_r, v_beta, W_ih, b_ih, W_hh, b_hh,
        W1a, b1a, W1b, b1b, W2a, b2a)

    base_bt = jnp.swapaxes(base_tb.reshape(T, B, DV), 0, 1).reshape(B, T * DV)

    sc = _make_sc()
    b2b_pad = jnp.concatenate([b2b, jnp.zeros((16 - NUM_D,), jnp.float32)])
    c_flat = sc(base_bt.reshape(-1), u_row.reshape(DV),
                W2b.reshape(NUM_D * DV), b2b_pad,
                c_seq.reshape(-1), d_seq.reshape(-1))
    C_seq = (c_flat.reshape(B, T, 4, NUM_D, 128)
             .transpose(0, 1, 2, 4, 3).reshape(B, T, NUM_C, NUM_D))

    alpha_seq = alpha_flat.reshape(B, T, NUM_D)
    return (alpha_seq, h_seq, C_seq)


# speedup vs baseline: 5.1422x; 1.2654x over previous
import functools

import jax
import jax.numpy as jnp
from jax import lax
from jax.experimental import pallas as pl
from jax.experimental.pallas import tpu as pltpu
from jax.experimental.pallas import tpu_sc as plsc

NUM_C = 512
NUM_D = 8
DV = 64
B = 16
T = 200
BT = B * T
HALF = NUM_C // 2
HW = HALF * NUM_D
PADW = HW

_HIGH = jax.lax.Precision.HIGHEST


def _dot(a, b):
    return jax.lax.dot_general(a, b, (((1,), (0,)), ((), ())),
                               precision=_HIGH)


def _setup_body(x_idx_smem, r_vmem, X_ref, vr_ref, vbeta_ref,
                WihT_ref, bih_ref, W2aLT_ref, W2aRT_ref, b2a_ref,
                gx_out, base_out, u_out, gin_ref):
    u_out[...] = _dot(vbeta_ref[...], W2aLT_ref[...])
    gin_ref[:, DV:] = r_vmem[...] * vr_ref[...]

    def gather_one(i, _):
        idx = x_idx_smem[i // B, i % B]
        gin_ref[pl.ds(i, 1), 0:DV] = X_ref[pl.ds(idx, 1), :]
        return 0

    jax.lax.fori_loop(0, BT, gather_one, 0, unroll=8)
    gx_out[...] = _dot(gin_ref[...], WihT_ref[...]) + bih_ref[...]
    base_out[...] = _dot(gin_ref[...], W2aRT_ref[...]) + b2a_ref[...]


def _run_setup(x_idx_T, r_T, X, v_r, v_beta, W_ih, b_ih, W2a, b2a):
    smem = pl.BlockSpec(memory_space=pltpu.MemorySpace.SMEM)
    anyv = pl.BlockSpec(memory_space=pltpu.MemorySpace.VMEM)
    return pl.pallas_call(
        _setup_body,
        in_specs=[smem] + [anyv] * 9,
        out_specs=[anyv, anyv, anyv],
        out_shape=[
            jax.ShapeDtypeStruct((BT, 3 * DV), jnp.float32),
            jax.ShapeDtypeStruct((BT, DV), jnp.float32),
            jax.ShapeDtypeStruct((1, DV), jnp.float32),
        ],
        scratch_shapes=[pltpu.VMEM((BT, 2 * DV), jnp.float32)],
    )(x_idx_T, r_T, X, v_r.reshape(1, DV), v_beta.reshape(1, DV),
      W_ih.T, b_ih.reshape(1, 3 * DV),
      W2a[:, :DV].T, W2a[:, DV:].T, b2a.reshape(1, DV))


def _gru_body(gx_ref, WhhT_ref, bhh_ref, W1aT_ref, b1a_ref, W1bT_ref,
              b1b_ref, alpha_out, h_out, h_ref):
    t0 = pl.program_id(0)

    @pl.when(t0 == 0)
    def _init():
        h_ref[...] = jnp.zeros_like(h_ref)

    h = h_ref[...]
    gh = _dot(h, WhhT_ref[...]) + bhh_ref[...]
    gx = gx_ref[pl.ds(t0 * B, B), :]
    r_g = jax.nn.sigmoid(gx[:, 0:DV] + gh[:, 0:DV])
    z_g = jax.nn.sigmoid(gx[:, DV:2 * DV] + gh[:, DV:2 * DV])
    n_g = jnp.tanh(gx[:, 2 * DV:] + r_g * gh[:, 2 * DV:])
    h_new = (1.0 - z_g) * n_g + z_g * h
    h_ref[...] = h_new
    h_out[:, pl.ds(t0, 1), :] = h_new.reshape(B, 1, DV)

    @pl.when(t0 == T - 1)
    def _alpha():
        h_flat = h_out[...].reshape(BT, DV)
        a1 = jnp.maximum(_dot(h_flat, W1aT_ref[...]) + b1a_ref[...], 0.0)
        alpha_out[...] = _dot(a1, W1bT_ref[...]) + b1b_ref[...]


def _run_gru(gx_all, W_hh, b_hh, W1a, b1a, W1b, b1b):
    anyv = pl.BlockSpec(memory_space=pltpu.MemorySpace.VMEM)
    grid_spec = pltpu.PrefetchScalarGridSpec(
        num_scalar_prefetch=0,
        grid=(T,),
        in_specs=[anyv] * 7,
        out_specs=[
            pl.BlockSpec((BT, NUM_D), lambda t: (0, 0)),
            pl.BlockSpec((B, T, DV), lambda t: (0, 0, 0)),
        ],
        scratch_shapes=[pltpu.VMEM((B, DV), jnp.float32)],
    )
    return pl.pallas_call(
        _gru_body,
        grid_spec=grid_spec,
        out_shape=[
            jax.ShapeDtypeStruct((BT, NUM_D), jnp.float32),
            jax.ShapeDtypeStruct((B, T, DV), jnp.float32),
        ],
        compiler_params=pltpu.CompilerParams(
            dimension_semantics=("arbitrary",)),
    )(gx_all, W_hh.T, b_hh.reshape(1, 3 * DV),
      W1a.T, b1a.reshape(1, DV), W1b.T, b1b.reshape(1, NUM_D))


_GDN = jax.lax.GatherDimensionNumbers(
    offset_dims=(), collapsed_slice_dims=(0,), start_index_map=(0,))


def _shuf(s, perm):
    return jax.lax.gather(
        s, perm[:, None], _GDN, slice_sizes=(1,),
        mode=jax.lax.GatherScatterMode.PROMISE_IN_BOUNDS)

def _sc_step(t, buf_cur, buf_oth, kc, c_v, d_v, base_v, u_v, w2b_v, b2b_vec,
             iota16, first):
    c_t = c_v[pl.ds(t, 16)][0]
    own = (c_t // HALF) == kc
    c_loc = c_t - kc * HALF
    p0 = (c_loc // 128) * 1024 + (c_loc % 128)
    a0 = p0 & ~15
    lane = p0 & 15

    if not first:
        c_p = c_v[pl.ds(t - 1, 16)][0]
        own_p = (c_p // HALF) == kc

        @pl.when(own_p)
        def _catch():
            cp_loc = c_p - kc * HALF
            b0 = ((cp_loc // 128) * 1024 + (cp_loc % 128)) & ~15
            for j in range(NUM_D):
                buf_cur[pl.ds(b0 + j * 128, 16)] =                     buf_oth[pl.ds(b0 + j * 128, 16)]

    @pl.when(own)
    def _update():
        d_t = d_v[pl.ds(t, 16)][0]
        bw = buf_cur[pl.ds(a0 + d_t * 128, 16)]
        beta = _shuf(bw, jnp.broadcast_to(lane, (16,)))
        acts = []
        for k in range(4):
            pre = (base_v[pl.ds(t * DV + k * 16, 16)]
                   + beta * u_v[pl.ds(k * 16, 16)])
            acts.append(jnp.maximum(pre, 0.0))
        for j in range(NUM_D):
            s = acts[0] * w2b_v[pl.ds(j * DV, 16)]
            for k in range(1, 4):
                s = s + acts[k] * w2b_v[pl.ds(j * DV + k * 16, 16)]
            for sh in (8, 4, 2, 1):
                s = s + _shuf(s, (iota16 + sh) % 16)
            nc_j = s + b2b_vec[j]
            w = buf_cur[pl.ds(a0 + j * 128, 16)]
            buf_cur[pl.ds(a0 + j * 128, 16)] = jnp.where(
                iota16 == lane, nc_j, w)


def _make_sc():
    mesh = plsc.VectorSubcoreMesh(core_axis_name="c", subcore_axis_name="s")

    @functools.partial(
        pl.kernel, mesh=mesh,
        out_type=jax.ShapeDtypeStruct((B * T * NUM_C * NUM_D,), jnp.float32),
        scratch_types=[
            pltpu.VMEM((PADW,), jnp.float32),
            pltpu.VMEM((PADW,), jnp.float32),
            pltpu.VMEM((T * DV,), jnp.float32),
            pltpu.VMEM((DV,), jnp.float32),
            pltpu.VMEM((NUM_D * DV,), jnp.float32),
            pltpu.VMEM((16,), jnp.float32),
            pltpu.VMEM((T + 16,), jnp.int32),
            pltpu.VMEM((T + 16,), jnp.int32),
            pltpu.SemaphoreType.DMA,
            pltpu.SemaphoreType.DMA,
        ],
    )
    def sc_kernel(base_hbm, u_hbm, w2b_hbm, b2b_hbm, c_hbm, d_hbm, out_hbm,
                  ping, pong, base_v, u_v, w2b_v, b2b_v, c_v, d_v,
                  semA, semB):
        kc = lax.axis_index("c")
        kb = lax.axis_index("s")
        pltpu.sync_copy(base_hbm.at[pl.ds(kb * (T * DV), T * DV)], base_v)
        pltpu.sync_copy(u_hbm, u_v)
        pltpu.sync_copy(w2b_hbm, w2b_v)
        pltpu.sync_copy(b2b_hbm, b2b_v)
        pltpu.sync_copy(c_hbm.at[pl.ds(kb * T, T)], c_v.at[pl.ds(0, T)])
        pltpu.sync_copy(d_hbm.at[pl.ds(kb * T, T)], d_v.at[pl.ds(0, T)])

        zeros16 = jnp.zeros((16,), jnp.float32)
        for k in range(PADW // 16):
            ping[pl.ds(k * 16, 16)] = zeros16
            pong[pl.ds(k * 16, 16)] = zeros16

        iota16 = jax.lax.broadcasted_iota(jnp.int32, (16,), 0)
        b2b_vec = b2b_v[pl.ds(0, 16)]
        args = (c_v, d_v, base_v, u_v, w2b_v, b2b_vec, iota16)
        off = kc * HW

        _sc_step(0, ping, pong, kc, *args, first=True)
        pltpu.async_copy(ping.at[pl.ds(0, HW)],
                         out_hbm.at[pl.ds((kb * T + (0)) * (NUM_C * NUM_D) + off, HW)], semA)
        _sc_step(1, pong, ping, kc, *args, first=False)
        pltpu.async_copy(pong.at[pl.ds(0, HW)],
                         out_hbm.at[pl.ds((kb * T + (1)) * (NUM_C * NUM_D) + off, HW)], semB)

        def body(i, _):
            t0 = 2 * i
            pltpu.make_async_copy(ping.at[pl.ds(0, HW)],
                                  out_hbm.at[pl.ds((kb * T + (t0)) * (NUM_C * NUM_D) + off, HW)],
                                  semA).wait()
            _sc_step(t0, ping, pong, kc, *args, first=False)
            pltpu.async_copy(ping.at[pl.ds(0, HW)],
                             out_hbm.at[pl.ds((kb * T + (t0)) * (NUM_C * NUM_D) + off, HW)], semA)
            pltpu.make_async_copy(pong.at[pl.ds(0, HW)],
                                  out_hbm.at[pl.ds((kb * T + (t0 + 1)) * (NUM_C * NUM_D) + off, HW)],
                                  semB).wait()
            _sc_step(t0 + 1, pong, ping, kc, *args, first=False)
            pltpu.async_copy(pong.at[pl.ds(0, HW)],
                             out_hbm.at[pl.ds((kb * T + (t0 + 1)) * (NUM_C * NUM_D) + off, HW)], semB)
            return 0

        lax.fori_loop(1, T // 2, body, 0)
        pltpu.make_async_copy(ping.at[pl.ds(0, HW)],
                              out_hbm.at[pl.ds((kb * T + (T - 2)) * (NUM_C * NUM_D) + off, HW)],
                              semA).wait()
        pltpu.make_async_copy(pong.at[pl.ds(0, HW)],
                              out_hbm.at[pl.ds((kb * T + (T - 1)) * (NUM_C * NUM_D) + off, HW)],
                              semB).wait()

    return sc_kernel


def kernel(c_seq, d_seq, r_seq, X, v_r, v_beta, W_ih, W_hh, b_ih, b_hh,
           W1a, b1a, W1b, b1b, W2a, b2a, W2b, b2b):
    c_seq = c_seq.astype(jnp.int32)
    d_seq = d_seq.astype(jnp.int32)
    x_idx_T = (c_seq + NUM_C * d_seq).T
    r_T = r_seq.T.reshape(BT, 1)

    gx_all, base_tb, u_row = _run_setup(
        x_idx_T, r_T, X, v_r, v_beta, W_ih, b_ih, W2a, b2a)

    base_bt = jnp.swapaxes(base_tb.reshape(T, B, DV), 0, 1).reshape(B, T * DV)

    sc = _make_sc()
    b2b_pad = jnp.concatenate([b2b, jnp.zeros((16 - NUM_D,), jnp.float32)])
    c_flat = sc(base_bt.reshape(-1), u_row.reshape(DV),
                W2b.reshape(NUM_D * DV), b2b_pad,
                c_seq.reshape(-1), d_seq.reshape(-1))

    alpha_flat, h_seq = _run_gru(gx_all, W_hh, b_hh, W1a, b1a, W1b, b1b)
    C_seq = (c_flat.reshape(B, T, 4, NUM_D, 128)
             .transpose(0, 1, 2, 4, 3).reshape(B, T, NUM_C, NUM_D))

    alpha_seq = alpha_flat.reshape(B, T, NUM_D)
    return (alpha_seq, h_seq, C_seq)


# speedup vs baseline: 5.3282x; 1.0362x over previous
import functools

import jax
import jax.numpy as jnp
from jax import lax
from jax.experimental import pallas as pl
from jax.experimental.pallas import tpu as pltpu
from jax.experimental.pallas import tpu_sc as plsc

NUM_C = 512
NUM_D = 8
DV = 64
B = 16
T = 200
BT = B * T
HALF = NUM_C // 2
HW = HALF * NUM_D
PADW = HW

_HIGH = jax.lax.Precision.HIGHEST


def _dot(a, b):
    return jax.lax.dot_general(a, b, (((1,), (0,)), ((), ())),
                               precision=_HIGH)


def _setup_body(x_idx_smem, r_vmem, X_ref, vr_ref, vbeta_ref,
                WihT_ref, bih_ref, W2aLT_ref, W2aRT_ref, b2a_ref,
                gx_out, base_out, u_out, gin_ref):
    u_out[...] = _dot(vbeta_ref[...], W2aLT_ref[...])
    gin_ref[:, DV:] = r_vmem[...] * vr_ref[...]

    def gather_one(i, _):
        idx = x_idx_smem[i // B, i % B]
        gin_ref[pl.ds(i, 1), 0:DV] = X_ref[pl.ds(idx, 1), :]
        return 0

    jax.lax.fori_loop(0, BT, gather_one, 0, unroll=8)
    gx_out[...] = _dot(gin_ref[...], WihT_ref[...]) + bih_ref[...]
    base_out[...] = _dot(gin_ref[...], W2aRT_ref[...]) + b2a_ref[...]


def _run_setup(x_idx_T, r_T, X, v_r, v_beta, W_ih, b_ih, W2a, b2a):
    smem = pl.BlockSpec(memory_space=pltpu.MemorySpace.SMEM)
    anyv = pl.BlockSpec(memory_space=pltpu.MemorySpace.VMEM)
    return pl.pallas_call(
        _setup_body,
        in_specs=[smem] + [anyv] * 9,
        out_specs=[anyv, anyv, anyv],
        out_shape=[
            jax.ShapeDtypeStruct((BT, 3 * DV), jnp.float32),
            jax.ShapeDtypeStruct((BT, DV), jnp.float32),
            jax.ShapeDtypeStruct((1, DV), jnp.float32),
        ],
        scratch_shapes=[pltpu.VMEM((BT, 2 * DV), jnp.float32)],
    )(x_idx_T, r_T, X, v_r.reshape(1, DV), v_beta.reshape(1, DV),
      W_ih.T, b_ih.reshape(1, 3 * DV),
      W2a[:, :DV].T, W2a[:, DV:].T, b2a.reshape(1, DV))


def _gru_body(gx_ref, WhhT_ref, bhh_ref, W1aT_ref, b1a_ref, W1bT_ref,
              b1b_ref, alpha_out, h_out, h_ref):
    t0 = pl.program_id(0)

    @pl.when(t0 == 0)
    def _init():
        h_ref[...] = jnp.zeros_like(h_ref)

    h = h_ref[...]
    gh = jax.lax.dot_general(h, WhhT_ref[...], (((1,), (0,)), ((), ())),
                             precision=jax.lax.Precision.DEFAULT) + bhh_ref[...]
    gx = gx_ref[pl.ds(t0 * B, B), :]
    r_g = jax.nn.sigmoid(gx[:, 0:DV] + gh[:, 0:DV])
    z_g = jax.nn.sigmoid(gx[:, DV:2 * DV] + gh[:, DV:2 * DV])
    n_g = jnp.tanh(gx[:, 2 * DV:] + r_g * gh[:, 2 * DV:])
    h_new = (1.0 - z_g) * n_g + z_g * h
    h_ref[...] = h_new
    h_out[:, pl.ds(t0, 1), :] = h_new.reshape(B, 1, DV)

    @pl.when(t0 == T - 1)
    def _alpha():
        h_flat = h_out[...].reshape(BT, DV)
        a1 = jnp.maximum(_dot(h_flat, W1aT_ref[...]) + b1a_ref[...], 0.0)
        alpha_out[...] = _dot(a1, W1bT_ref[...]) + b1b_ref[...]


def _run_gru(gx_all, W_hh, b_hh, W1a, b1a, W1b, b1b):
    anyv = pl.BlockSpec(memory_space=pltpu.MemorySpace.VMEM)
    grid_spec = pltpu.PrefetchScalarGridSpec(
        num_scalar_prefetch=0,
        grid=(T,),
        in_specs=[anyv] * 7,
        out_specs=[
            pl.BlockSpec((BT, NUM_D), lambda t: (0, 0)),
            pl.BlockSpec((B, T, DV), lambda t: (0, 0, 0)),
        ],
        scratch_shapes=[pltpu.VMEM((B, DV), jnp.float32)],
    )
    return pl.pallas_call(
        _gru_body,
        grid_spec=grid_spec,
        out_shape=[
            jax.ShapeDtypeStruct((BT, NUM_D), jnp.float32),
            jax.ShapeDtypeStruct((B, T, DV), jnp.float32),
        ],
        compiler_params=pltpu.CompilerParams(
            dimension_semantics=("arbitrary",)),
    )(gx_all, W_hh.T, b_hh.reshape(1, 3 * DV),
      W1a.T, b1a.reshape(1, DV), W1b.T, b1b.reshape(1, NUM_D))


_GDN = jax.lax.GatherDimensionNumbers(
    offset_dims=(), collapsed_slice_dims=(0,), start_index_map=(0,))


def _shuf(s, perm):
    return jax.lax.gather(
        s, perm[:, None], _GDN, slice_sizes=(1,),
        mode=jax.lax.GatherScatterMode.PROMISE_IN_BOUNDS)

def _sc_step(t, buf_cur, buf_oth, kc, c_v, d_v, base_v, u_v, w2b_v, b2b_vec,
             iota16, first):
    c_t = c_v[pl.ds(t, 16)][0]
    own = (c_t // HALF) == kc
    c_loc = c_t - kc * HALF
    p0 = (c_loc // 128) * 1024 + (c_loc % 128)
    a0 = p0 & ~15
    lane = p0 & 15

    if not first:
        c_p = c_v[pl.ds(t - 1, 16)][0]
        own_p = (c_p // HALF) == kc

        @pl.when(own_p)
        def _catch():
            cp_loc = c_p - kc * HALF
            b0 = ((cp_loc // 128) * 1024 + (cp_loc % 128)) & ~15
            for j in range(NUM_D):
                buf_cur[pl.ds(b0 + j * 128, 16)] =                     buf_oth[pl.ds(b0 + j * 128, 16)]

    @pl.when(own)
    def _update():
        d_t = d_v[pl.ds(t, 16)][0]
        bw = buf_cur[pl.ds(a0 + d_t * 128, 16)]
        beta = _shuf(bw, jnp.broadcast_to(lane, (16,)))
        acts = []
        for k in range(4):
            pre = (base_v[pl.ds(t * DV + k * 16, 16)]
                   + beta * u_v[pl.ds(k * 16, 16)])
            acts.append(jnp.maximum(pre, 0.0))
        for j in range(NUM_D):
            s = acts[0] * w2b_v[pl.ds(j * DV, 16)]
            for k in range(1, 4):
                s = s + acts[k] * w2b_v[pl.ds(j * DV + k * 16, 16)]
            for sh in (8, 4, 2, 1):
                s = s + _shuf(s, (iota16 + sh) % 16)
            nc_j = s + b2b_vec[j]
            w = buf_cur[pl.ds(a0 + j * 128, 16)]
            buf_cur[pl.ds(a0 + j * 128, 16)] = jnp.where(
                iota16 == lane, nc_j, w)


def _make_sc():
    mesh = plsc.VectorSubcoreMesh(core_axis_name="c", subcore_axis_name="s")

    @functools.partial(
        pl.kernel, mesh=mesh,
        out_type=jax.ShapeDtypeStruct((B * T * NUM_C * NUM_D,), jnp.float32),
        scratch_types=[
            pltpu.VMEM((PADW,), jnp.float32),
            pltpu.VMEM((PADW,), jnp.float32),
            pltpu.VMEM((T * DV,), jnp.float32),
            pltpu.VMEM((DV,), jnp.float32),
            pltpu.VMEM((NUM_D * DV,), jnp.float32),
            pltpu.VMEM((16,), jnp.float32),
            pltpu.VMEM((T + 16,), jnp.int32),
            pltpu.VMEM((T + 16,), jnp.int32),
            pltpu.SemaphoreType.DMA,
            pltpu.SemaphoreType.DMA,
        ],
    )
    def sc_kernel(base_hbm, u_hbm, w2b_hbm, b2b_hbm, c_hbm, d_hbm, out_hbm,
                  ping, pong, base_v, u_v, w2b_v, b2b_v, c_v, d_v,
                  semA, semB):
        kc = lax.axis_index("c")
        kb = lax.axis_index("s")
        pltpu.sync_copy(base_hbm.at[pl.ds(kb * (T * DV), T * DV)], base_v)
        pltpu.sync_copy(u_hbm, u_v)
        pltpu.sync_copy(w2b_hbm, w2b_v)
        pltpu.sync_copy(b2b_hbm, b2b_v)
        pltpu.sync_copy(c_hbm.at[pl.ds(kb * T, T)], c_v.at[pl.ds(0, T)])
        pltpu.sync_copy(d_hbm.at[pl.ds(kb * T, T)], d_v.at[pl.ds(0, T)])

        zeros16 = jnp.zeros((16,), jnp.float32)
        for k in range(PADW // 16):
            ping[pl.ds(k * 16, 16)] = zeros16
            pong[pl.ds(k * 16, 16)] = zeros16

        iota16 = jax.lax.broadcasted_iota(jnp.int32, (16,), 0)
        b2b_vec = b2b_v[pl.ds(0, 16)]
        args = (c_v, d_v, base_v, u_v, w2b_v, b2b_vec, iota16)
        off = kc * HW

        _sc_step(0, ping, pong, kc, *args, first=True)
        pltpu.async_copy(ping.at[pl.ds(0, HW)],
                         out_hbm.at[pl.ds((kb * T + (0)) * (NUM_C * NUM_D) + off, HW)], semA)
        _sc_step(1, pong, ping, kc, *args, first=False)
        pltpu.async_copy(pong.at[pl.ds(0, HW)],
                         out_hbm.at[pl.ds((kb * T + (1)) * (NUM_C * NUM_D) + off, HW)], semB)

        def body(i, _):
            t0 = 2 * i
            pltpu.make_async_copy(ping.at[pl.ds(0, HW)],
                                  out_hbm.at[pl.ds((kb * T + (t0)) * (NUM_C * NUM_D) + off, HW)],
                                  semA).wait()
            _sc_step(t0, ping, pong, kc, *args, first=False)
            pltpu.async_copy(ping.at[pl.ds(0, HW)],
                             out_hbm.at[pl.ds((kb * T + (t0)) * (NUM_C * NUM_D) + off, HW)], semA)
            pltpu.make_async_copy(pong.at[pl.ds(0, HW)],
                                  out_hbm.at[pl.ds((kb * T + (t0 + 1)) * (NUM_C * NUM_D) + off, HW)],
                                  semB).wait()
            _sc_step(t0 + 1, pong, ping, kc, *args, first=False)
            pltpu.async_copy(pong.at[pl.ds(0, HW)],
                             out_hbm.at[pl.ds((kb * T + (t0 + 1)) * (NUM_C * NUM_D) + off, HW)], semB)
            return 0

        lax.fori_loop(1, T // 2, body, 0)
        pltpu.make_async_copy(ping.at[pl.ds(0, HW)],
                              out_hbm.at[pl.ds((kb * T + (T - 2)) * (NUM_C * NUM_D) + off, HW)],
                              semA).wait()
        pltpu.make_async_copy(pong.at[pl.ds(0, HW)],
                              out_hbm.at[pl.ds((kb * T + (T - 1)) * (NUM_C * NUM_D) + off, HW)],
                              semB).wait()

    return sc_kernel


def kernel(c_seq, d_seq, r_seq, X, v_r, v_beta, W_ih, W_hh, b_ih, b_hh,
           W1a, b1a, W1b, b1b, W2a, b2a, W2b, b2b):
    c_seq = c_seq.astype(jnp.int32)
    d_seq = d_seq.astype(jnp.int32)
    x_idx_T = (c_seq + NUM_C * d_seq).T
    r_T = r_seq.T.reshape(BT, 1)

    gx_all, base_tb, u_row = _run_setup(
        x_idx_T, r_T, X, v_r, v_beta, W_ih, b_ih, W2a, b2a)

    base_bt = jnp.swapaxes(base_tb.reshape(T, B, DV), 0, 1).reshape(B, T * DV)

    sc = _make_sc()
    b2b_pad = jnp.concatenate([b2b, jnp.zeros((16 - NUM_D,), jnp.float32)])
    c_flat = sc(base_bt.reshape(-1), u_row.reshape(DV),
                W2b.reshape(NUM_D * DV), b2b_pad,
                c_seq.reshape(-1), d_seq.reshape(-1))

    alpha_flat, h_seq = _run_gru(gx_all, W_hh, b_hh, W1a, b1a, W1b, b1b)
    C_seq = (c_flat.reshape(B, T, 4, NUM_D, 128)
             .transpose(0, 1, 2, 4, 3).reshape(B, T, NUM_C, NUM_D))

    alpha_seq = alpha_flat.reshape(B, T, NUM_D)
    return (alpha_seq, h_seq, C_seq)


# speedup vs baseline: 5.3811x; 1.0099x over previous
import functools

import jax
import jax.numpy as jnp
from jax import lax
from jax.experimental import pallas as pl
from jax.experimental.pallas import tpu as pltpu
from jax.experimental.pallas import tpu_sc as plsc

NUM_C = 512
NUM_D = 8
DV = 64
B = 16
T = 200
BT = B * T
HALF = NUM_C // 2
HW = HALF * NUM_D
PADW = HW

_HIGH = jax.lax.Precision.HIGHEST


def _dot(a, b):
    return jax.lax.dot_general(a, b, (((1,), (0,)), ((), ())),
                               precision=_HIGH)


def _setup_body(x_idx_smem, r_vmem, X_ref, vr_ref, vbeta_ref,
                WihT_ref, bih_ref, W2aLT_ref, W2aRT_ref, b2a_ref,
                gx_out, base_out, u_out, gin_ref):
    u_out[...] = _dot(vbeta_ref[...], W2aLT_ref[...])
    gin_ref[:, DV:] = r_vmem[...] * vr_ref[...]

    def gather_one(i, _):
        idx = x_idx_smem[i // B, i % B]
        gin_ref[pl.ds(i, 1), 0:DV] = X_ref[pl.ds(idx, 1), :]
        return 0

    jax.lax.fori_loop(0, BT, gather_one, 0, unroll=32)
    gx_out[...] = _dot(gin_ref[...], WihT_ref[...]) + bih_ref[...]
    base_out[...] = _dot(gin_ref[...], W2aRT_ref[...]) + b2a_ref[...]


def _run_setup(x_idx_T, r_T, X, v_r, v_beta, W_ih, b_ih, W2a, b2a):
    smem = pl.BlockSpec(memory_space=pltpu.MemorySpace.SMEM)
    anyv = pl.BlockSpec(memory_space=pltpu.MemorySpace.VMEM)
    return pl.pallas_call(
        _setup_body,
        in_specs=[smem] + [anyv] * 9,
        out_specs=[anyv, anyv, anyv],
        out_shape=[
            jax.ShapeDtypeStruct((BT, 3 * DV), jnp.float32),
            jax.ShapeDtypeStruct((BT, DV), jnp.float32),
            jax.ShapeDtypeStruct((1, DV), jnp.float32),
        ],
        scratch_shapes=[pltpu.VMEM((BT, 2 * DV), jnp.float32)],
    )(x_idx_T, r_T, X, v_r.reshape(1, DV), v_beta.reshape(1, DV),
      W_ih.T, b_ih.reshape(1, 3 * DV),
      W2a[:, :DV].T, W2a[:, DV:].T, b2a.reshape(1, DV))


def _gru_body(gx_ref, WhhT_ref, bhh_ref, W1aT_ref, b1a_ref, W1bT_ref,
              b1b_ref, alpha_out, h_out, h_ref):
    t0 = pl.program_id(0)

    @pl.when(t0 == 0)
    def _init():
        h_ref[...] = jnp.zeros_like(h_ref)

    h = h_ref[...]
    gh = jax.lax.dot_general(h, WhhT_ref[...], (((1,), (0,)), ((), ())),
                             precision=jax.lax.Precision.DEFAULT) + bhh_ref[...]
    gx = gx_ref[pl.ds(t0 * B, B), :]
    r_g = jax.nn.sigmoid(gx[:, 0:DV] + gh[:, 0:DV])
    z_g = jax.nn.sigmoid(gx[:, DV:2 * DV] + gh[:, DV:2 * DV])
    n_g = jnp.tanh(gx[:, 2 * DV:] + r_g * gh[:, 2 * DV:])
    h_new = (1.0 - z_g) * n_g + z_g * h
    h_ref[...] = h_new
    h_out[:, pl.ds(t0, 1), :] = h_new.reshape(B, 1, DV)

    @pl.when(t0 == T - 1)
    def _alpha():
        h_flat = h_out[...].reshape(BT, DV)
        a1 = jnp.maximum(_dot(h_flat, W1aT_ref[...]) + b1a_ref[...], 0.0)
        alpha_out[...] = _dot(a1, W1bT_ref[...]) + b1b_ref[...]


def _run_gru(gx_all, W_hh, b_hh, W1a, b1a, W1b, b1b):
    anyv = pl.BlockSpec(memory_space=pltpu.MemorySpace.VMEM)
    grid_spec = pltpu.PrefetchScalarGridSpec(
        num_scalar_prefetch=0,
        grid=(T,),
        in_specs=[anyv] * 7,
        out_specs=[
            pl.BlockSpec((BT, NUM_D), lambda t: (0, 0)),
            pl.BlockSpec((B, T, DV), lambda t: (0, 0, 0)),
        ],
        scratch_shapes=[pltpu.VMEM((B, DV), jnp.float32)],
    )
    return pl.pallas_call(
        _gru_body,
        grid_spec=grid_spec,
        out_shape=[
            jax.ShapeDtypeStruct((BT, NUM_D), jnp.float32),
            jax.ShapeDtypeStruct((B, T, DV), jnp.float32),
        ],
        compiler_params=pltpu.CompilerParams(
            dimension_semantics=("arbitrary",)),
    )(gx_all, W_hh.T, b_hh.reshape(1, 3 * DV),
      W1a.T, b1a.reshape(1, DV), W1b.T, b1b.reshape(1, NUM_D))


_GDN = jax.lax.GatherDimensionNumbers(
    offset_dims=(), collapsed_slice_dims=(0,), start_index_map=(0,))


def _shuf(s, perm):
    return jax.lax.gather(
        s, perm[:, None], _GDN, slice_sizes=(1,),
        mode=jax.lax.GatherScatterMode.PROMISE_IN_BOUNDS)

def _sc_step(t, buf_cur, buf_oth, kc, c_v, d_v, base_v, u_v, w2b_v, b2b_vec,
             iota16, first):
    c_t = c_v[pl.ds(t, 16)][0]
    own = (c_t // HALF) == kc
    c_loc = c_t - kc * HALF
    p0 = (c_loc // 128) * 1024 + (c_loc % 128)
    a0 = p0 & ~15
    lane = p0 & 15

    if not first:
        c_p = c_v[pl.ds(t - 1, 16)][0]
        own_p = (c_p // HALF) == kc

        @pl.when(own_p)
        def _catch():
            cp_loc = c_p - kc * HALF
            b0 = ((cp_loc // 128) * 1024 + (cp_loc % 128)) & ~15
            for j in range(NUM_D):
                buf_cur[pl.ds(b0 + j * 128, 16)] =                     buf_oth[pl.ds(b0 + j * 128, 16)]

    @pl.when(own)
    def _update():
        d_t = d_v[pl.ds(t, 16)][0]
        bw = buf_cur[pl.ds(a0 + d_t * 128, 16)]
        beta = _shuf(bw, jnp.broadcast_to(lane, (16,)))
        acts = []
        for k in range(4):
            pre = (base_v[pl.ds(t * DV + k * 16, 16)]
                   + beta * u_v[pl.ds(k * 16, 16)])
            acts.append(jnp.maximum(pre, 0.0))
        for j in range(NUM_D):
            s = acts[0] * w2b_v[pl.ds(j * DV, 16)]
            for k in range(1, 4):
                s = s + acts[k] * w2b_v[pl.ds(j * DV + k * 16, 16)]
            for sh in (8, 4, 2, 1):
                s = s + _shuf(s, (iota16 + sh) % 16)
            nc_j = s + b2b_vec[j]
            w = buf_cur[pl.ds(a0 + j * 128, 16)]
            buf_cur[pl.ds(a0 + j * 128, 16)] = jnp.where(
                iota16 == lane, nc_j, w)


def _make_sc():
    mesh = plsc.VectorSubcoreMesh(core_axis_name="c", subcore_axis_name="s")

    @functools.partial(
        pl.kernel, mesh=mesh,
        out_type=jax.ShapeDtypeStruct((B * T * NUM_C * NUM_D,), jnp.float32),
        scratch_types=[
            pltpu.VMEM((PADW,), jnp.float32),
            pltpu.VMEM((PADW,), jnp.float32),
            pltpu.VMEM((T * DV,), jnp.float32),
            pltpu.VMEM((DV,), jnp.float32),
            pltpu.VMEM((NUM_D * DV,), jnp.float32),
            pltpu.VMEM((16,), jnp.float32),
            pltpu.VMEM((T + 16,), jnp.int32),
            pltpu.VMEM((T + 16,), jnp.int32),
            pltpu.SemaphoreType.DMA,
            pltpu.SemaphoreType.DMA,
        ],
    )
    def sc_kernel(base_hbm, u_hbm, w2b_hbm, b2b_hbm, c_hbm, d_hbm, out_hbm,
                  ping, pong, base_v, u_v, w2b_v, b2b_v, c_v, d_v,
                  semA, semB):
        kc = lax.axis_index("c")
        kb = lax.axis_index("s")
        pltpu.sync_copy(base_hbm.at[pl.ds(kb * (T * DV), T * DV)], base_v)
        pltpu.sync_copy(u_hbm, u_v)
        pltpu.sync_copy(w2b_hbm, w2b_v)
        pltpu.sync_copy(b2b_hbm, b2b_v)
        pltpu.sync_copy(c_hbm.at[pl.ds(kb * T, T)], c_v.at[pl.ds(0, T)])
        pltpu.sync_copy(d_hbm.at[pl.ds(kb * T, T)], d_v.at[pl.ds(0, T)])

        zeros16 = jnp.zeros((16,), jnp.float32)
        for k in range(PADW // 16):
            ping[pl.ds(k * 16, 16)] = zeros16
            pong[pl.ds(k * 16, 16)] = zeros16

        iota16 = jax.lax.broadcasted_iota(jnp.int32, (16,), 0)
        b2b_vec = b2b_v[pl.ds(0, 16)]
        args = (c_v, d_v, base_v, u_v, w2b_v, b2b_vec, iota16)
        off = kc * HW

        _sc_step(0, ping, pong, kc, *args, first=True)
        pltpu.async_copy(ping.at[pl.ds(0, HW)],
                         out_hbm.at[pl.ds((kb * T + (0)) * (NUM_C * NUM_D) + off, HW)], semA)
        _sc_step(1, pong, ping, kc, *args, first=False)
        pltpu.async_copy(pong.at[pl.ds(0, HW)],
                         out_hbm.at[pl.ds((kb * T + (1)) * (NUM_C * NUM_D) + off, HW)], semB)

        def body(i, _):
            t0 = 2 * i
            pltpu.make_async_copy(ping.at[pl.ds(0, HW)],
                                  out_hbm.at[pl.ds((kb * T + (t0)) * (NUM_C * NUM_D) + off, HW)],
                                  semA).wait()
            _sc_step(t0, ping, pong, kc, *args, first=False)
            pltpu.async_copy(ping.at[pl.ds(0, HW)],
                             out_hbm.at[pl.ds((kb * T + (t0)) * (NUM_C * NUM_D) + off, HW)], semA)
            pltpu.make_async_copy(pong.at[pl.ds(0, HW)],
                                  out_hbm.at[pl.ds((kb * T + (t0 + 1)) * (NUM_C * NUM_D) + off, HW)],
                                  semB).wait()
            _sc_step(t0 + 1, pong, ping, kc, *args, first=False)
            pltpu.async_copy(pong.at[pl.ds(0, HW)],
                             out_hbm.at[pl.ds((kb * T + (t0 + 1)) * (NUM_C * NUM_D) + off, HW)], semB)
            return 0

        lax.fori_loop(1, T // 2, body, 0)
        pltpu.make_async_copy(ping.at[pl.ds(0, HW)],
                              out_hbm.at[pl.ds((kb * T + (T - 2)) * (NUM_C * NUM_D) + off, HW)],
                              semA).wait()
        pltpu.make_async_copy(pong.at[pl.ds(0, HW)],
                              out_hbm.at[pl.ds((kb * T + (T - 1)) * (NUM_C * NUM_D) + off, HW)],
                              semB).wait()

    return sc_kernel


def kernel(c_seq, d_seq, r_seq, X, v_r, v_beta, W_ih, W_hh, b_ih, b_hh,
           W1a, b1a, W1b, b1b, W2a, b2a, W2b, b2b):
    c_seq = c_seq.astype(jnp.int32)
    d_seq = d_seq.astype(jnp.int32)
    x_idx_T = (c_seq + NUM_C * d_seq).T
    r_T = r_seq.T.reshape(BT, 1)

    gx_all, base_tb, u_row = _run_setup(
        x_idx_T, r_T, X, v_r, v_beta, W_ih, b_ih, W2a, b2a)

    base_bt = jnp.swapaxes(base_tb.reshape(T, B, DV), 0, 1).reshape(B, T * DV)

    sc = _make_sc()
    b2b_pad = jnp.concatenate([b2b, jnp.zeros((16 - NUM_D,), jnp.float32)])
    c_flat = sc(base_bt.reshape(-1), u_row.reshape(DV),
                W2b.reshape(NUM_D * DV), b2b_pad,
                c_seq.reshape(-1), d_seq.reshape(-1))

    alpha_flat, h_seq = _run_gru(gx_all, W_hh, b_hh, W1a, b1a, W1b, b1b)
    C_seq = (c_flat.reshape(B, T, 4, NUM_D, 128)
             .transpose(0, 1, 2, 4, 3).reshape(B, T, NUM_C, NUM_D))

    alpha_seq = alpha_flat.reshape(B, T, NUM_D)
    return (alpha_seq, h_seq, C_seq)
